# baseline (speedup 1.0000x reference)
import jax
import jax.numpy as jnp
from jax.experimental import pallas as pl
from jax.experimental.pallas import tpu as pltpu

_BM = 256


def _gcn7(x_ref, w_ref, b_ref, adj_l_ref, adj_r_ref, out_ref, support_ref):
    i = pl.program_id(0)
    nh = support_ref.shape[0] // 2

    @pl.when(i == 0)
    def _support():
        support_ref[...] = jnp.dot(
            x_ref[...], w_ref[...], preferred_element_type=jnp.float32
        )

    acc = jnp.dot(
        adj_l_ref[...], support_ref[pl.ds(0, nh), :],
        preferred_element_type=jnp.float32,
    ) + jnp.dot(
        adj_r_ref[...], support_ref[pl.ds(nh, nh), :],
        preferred_element_type=jnp.float32,
    )
    x_blk = x_ref[pl.ds(i * _BM, _BM), :]
    out_ref[...] = jnp.tanh(acc + b_ref[...] + x_blk)


def kernel(x, adj, W, b):
    n, d = x.shape
    b2 = b.reshape(1, d)
    nh = n // 2
    return pl.pallas_call(
        _gcn7,
        grid=(n // _BM,),
        in_specs=[
            pl.BlockSpec((n, d), lambda i: (0, 0)),
            pl.BlockSpec((d, d), lambda i: (0, 0)),
            pl.BlockSpec((1, d), lambda i: (0, 0)),
            pl.BlockSpec((_BM, nh), lambda i: (i, 0)),
            pl.BlockSpec((_BM, nh), lambda i: (i, 1)),
        ],
        out_specs=pl.BlockSpec((_BM, d), lambda i: (i, 0)),
        out_shape=jax.ShapeDtypeStruct((n, d), jnp.float32),
        scratch_shapes=[pltpu.VMEM((n, d), jnp.float32)],
        compiler_params=pltpu.CompilerParams(
            dimension_semantics=("arbitrary",),
        ),
    )(x, W, b2, adj, adj)
